# Initial kernel scaffold; baseline (speedup 1.0000x reference)
import jax
import jax.numpy as jnp
from jax import lax
from jax.experimental import pallas as pl
from jax.experimental.pallas import tpu as pltpu
from jax.experimental.pallas import tpu_sc as plsc

N = 10000
E = 320000
D = 128

NC = 2
NS = 16
NW = NC * NS
L = 16

N_PAD = 10240
ROWS_PER_TILE = N_PAD // NS
CHUNK = 128
PER_W = ((E + NW * CHUNK - 1) // (NW * CHUNK)) * CHUNK
E_PAD = PER_W * NW
N_CHUNKS = PER_W // CHUNK


def _make_seg_sum(with_deg: bool):
    mesh = plsc.VectorSubcoreMesh(core_axis_name="c", subcore_axis_name="s")
    out_type = [jax.ShapeDtypeStruct((NC, N_PAD, D), jnp.float32)]
    scratch = [
        pltpu.VMEM_SHARED((N_PAD, D), jnp.float32),
        pltpu.VMEM((CHUNK,), jnp.int32),
        pltpu.VMEM((CHUNK,), jnp.int32),
        pltpu.VMEM((CHUNK, D), jnp.float32),
        pltpu.SemaphoreType.DMA,
    ]
    if with_deg:
        out_type.append(jax.ShapeDtypeStruct((NC, N_PAD, L), jnp.float32))
        scratch += [
            pltpu.VMEM_SHARED((N_PAD, L), jnp.float32),
            pltpu.VMEM((CHUNK, L), jnp.float32),
        ]

    def body(*refs):
        if with_deg:
            (table, src, dst, zeros, ones,
             out, degout,
             acc, src_v, dst_v, rows_v, sem, deg_sh, ones_v) = refs
        else:
            (table, src, dst, zeros,
             out,
             acc, src_v, dst_v, rows_v, sem) = refs
        c = lax.axis_index("c")
        s = lax.axis_index("s")
        wid = s * NC + c
        r0 = s * ROWS_PER_TILE

        pltpu.sync_copy(zeros.at[pl.ds(r0, ROWS_PER_TILE)],
                        acc.at[pl.ds(r0, ROWS_PER_TILE)])
        if with_deg:
            pltpu.sync_copy(zeros.at[pl.ds(r0, ROWS_PER_TILE), pl.ds(0, L)],
                            deg_sh.at[pl.ds(r0, ROWS_PER_TILE)])
            pltpu.sync_copy(ones, ones_v)
        plsc.subcore_barrier()

        base0 = wid * PER_W

        def chunk(i, carry):
            base = base0 + i * CHUNK
            pltpu.sync_copy(src.at[pl.ds(base, CHUNK)], src_v)
            pltpu.sync_copy(dst.at[pl.ds(base, CHUNK)], dst_v)
            pltpu.async_copy(table.at[src_v], rows_v, sem).wait()
            pltpu.sync_copy(rows_v, acc.at[dst_v], add=True)
            if with_deg:
                pltpu.sync_copy(ones_v, deg_sh.at[dst_v], add=True)
            return carry

        lax.fori_loop(0, N_CHUNKS, chunk, 0)
        plsc.subcore_barrier()

        pltpu.sync_copy(acc.at[pl.ds(r0, ROWS_PER_TILE)],
                        out.at[c, pl.ds(r0, ROWS_PER_TILE)])
        if with_deg:
            pltpu.sync_copy(deg_sh.at[pl.ds(r0, ROWS_PER_TILE)],
                            degout.at[c, pl.ds(r0, ROWS_PER_TILE)])

    return pl.kernel(body, out_type=tuple(out_type) if with_deg else out_type[0],
                     mesh=mesh, scratch_types=scratch)


_seg_sum_deg = _make_seg_sum(True)
_seg_sum = _make_seg_sum(False)

BLK = 512


def _tc_layer(parts, degparts, xin, WlT, bl, WrT):
    def body(p_ref, dg_ref, x_ref, wl_ref, bl_ref, wr_ref, o_ref):
        agg = p_ref[0] + p_ref[1]
        deg = dg_ref[0, :, 0:1] + dg_ref[1, :, 0:1]
        mean = agg / jnp.maximum(deg, 1.0)
        h = (jnp.dot(mean, wl_ref[...], preferred_element_type=jnp.float32)
             + bl_ref[...]
             + jnp.dot(x_ref[...], wr_ref[...], preferred_element_type=jnp.float32))
        o_ref[...] = jnp.maximum(h, 0.0)

    return pl.pallas_call(
        body,
        grid=(N_PAD // BLK,),
        in_specs=[
            pl.BlockSpec((2, BLK, D), lambda i: (0, i, 0)),
            pl.BlockSpec((2, BLK, L), lambda i: (0, i, 0)),
            pl.BlockSpec((BLK, D), lambda i: (i, 0)),
            pl.BlockSpec((D, D), lambda i: (0, 0)),
            pl.BlockSpec((1, D), lambda i: (0, 0)),
            pl.BlockSpec((D, D), lambda i: (0, 0)),
        ],
        out_specs=pl.BlockSpec((BLK, D), lambda i: (i, 0)),
        out_shape=jax.ShapeDtypeStruct((N_PAD, D), jnp.float32),
    )(parts, degparts, xin, WlT, bl, WrT)


def _tc_layer2_heads(parts, degparts, hin, WlT, bl, WrT, WpT, bp, WdT, bd):
    def body(p_ref, dg_ref, h_ref, wl_ref, bl_ref, wr_ref,
             wp_ref, bp_ref, wd_ref, bd_ref, lo_ref, up_ref):
        agg = p_ref[0] + p_ref[1]
        deg = dg_ref[0, :, 0:1] + dg_ref[1, :, 0:1]
        mean = agg / jnp.maximum(deg, 1.0)
        h2 = (jnp.dot(mean, wl_ref[...], preferred_element_type=jnp.float32)
              + bl_ref[...]
              + jnp.dot(h_ref[...], wr_ref[...], preferred_element_type=jnp.float32))
        h2 = jnp.maximum(h2, 0.0)
        p = jnp.dot(h2, wp_ref[...], preferred_element_type=jnp.float32) + bp_ref[...]
        dz = jnp.dot(h2, wd_ref[...], preferred_element_type=jnp.float32) + bd_ref[...]
        sg = 1.0 / (1.0 + jnp.exp(-dz))
        lo_ref[...] = p - sg
        up_ref[...] = p + sg

    return pl.pallas_call(
        body,
        grid=(N_PAD // BLK,),
        in_specs=[
            pl.BlockSpec((2, BLK, D), lambda i: (0, i, 0)),
            pl.BlockSpec((2, BLK, L), lambda i: (0, i, 0)),
            pl.BlockSpec((BLK, D), lambda i: (i, 0)),
            pl.BlockSpec((D, D), lambda i: (0, 0)),
            pl.BlockSpec((1, D), lambda i: (0, 0)),
            pl.BlockSpec((D, D), lambda i: (0, 0)),
            pl.BlockSpec((D, 1), lambda i: (0, 0)),
            pl.BlockSpec((1, 1), lambda i: (0, 0)),
            pl.BlockSpec((D, 1), lambda i: (0, 0)),
            pl.BlockSpec((1, 1), lambda i: (0, 0)),
        ],
        out_specs=[
            pl.BlockSpec((BLK, 1), lambda i: (i, 0)),
            pl.BlockSpec((BLK, 1), lambda i: (i, 0)),
        ],
        out_shape=[
            jax.ShapeDtypeStruct((N_PAD, 1), jnp.float32),
            jax.ShapeDtypeStruct((N_PAD, 1), jnp.float32),
        ],
    )(parts, degparts, hin, WlT, bl, WrT, WpT, bp, WdT, bd)


def kernel(x, edge_index, W1l, b1l, W1r, W2l, b2l, W2r, Wp, bp, Wd, bd):
    src = edge_index[0]
    dst = edge_index[1]
    pad_n = E_PAD - E
    pad_ids = jnp.arange(pad_n, dtype=jnp.int32)
    src_p = jnp.concatenate([src, pad_ids % N])
    dst_p = jnp.concatenate([dst, N + pad_ids % (N_PAD - N)])
    x_pad = jnp.zeros((N_PAD, D), jnp.float32).at[:N].set(x)
    zeros = jnp.zeros((N_PAD, D), jnp.float32)
    ones = jnp.ones((CHUNK, L), jnp.float32)

    parts1, degp = _seg_sum_deg(x_pad, src_p, dst_p, zeros, ones)
    h1 = _tc_layer(parts1, degp, x_pad, W1l.T, b1l.reshape(1, D), W1r.T)
    parts2 = _seg_sum(h1, src_p, dst_p, zeros)
    lo, up = _tc_layer2_heads(parts2, degp, h1, W2l.T, b2l.reshape(1, D),
                              W2r.T, Wp.T, bp.reshape(1, 1),
                              Wd.T, bd.reshape(1, 1))
    return lo[:N], up[:N]


# speedup vs baseline: 5.2770x; 5.2770x over previous
import jax
import jax.numpy as jnp
from jax import lax
from jax.experimental import pallas as pl
from jax.experimental.pallas import tpu as pltpu
from jax.experimental.pallas import tpu_sc as plsc

N = 10000
E = 320000
D = 128

NC = 2
NS = 16
NW = NC * NS
L = 16

N_PAD = 10240
ROWS_PER_TILE = N_PAD // NS
CHUNK = 128
PER_W = ((E + NW * CHUNK - 1) // (NW * CHUNK)) * CHUNK
E_PAD = PER_W * NW
N_CHUNKS = PER_W // CHUNK

_mesh = plsc.VectorSubcoreMesh(core_axis_name="c", subcore_axis_name="s")


def _seg_sum(table, src, dst, onesz):
    def body(table, src, dst, onesz, out, acc, src_v, dst_v, rows_v, sem):
        c = lax.axis_index("c")
        s = lax.axis_index("s")
        wid = s * NC + c
        r0 = s * ROWS_PER_TILE
        base0 = wid * PER_W

        for k in range(ROWS_PER_TILE // CHUNK):
            pltpu.sync_copy(onesz.at[pl.ds(CHUNK, CHUNK)], rows_v)
            pltpu.sync_copy(rows_v, acc.at[pl.ds(r0 + k * CHUNK, CHUNK)])
        plsc.subcore_barrier()

        def chunk(i, carry):
            base = base0 + i * CHUNK
            pltpu.sync_copy(src.at[pl.ds(base, CHUNK)], src_v)
            pltpu.sync_copy(dst.at[pl.ds(base, CHUNK)], dst_v)
            pltpu.async_copy(table.at[src_v], rows_v, sem).wait()
            pltpu.sync_copy(rows_v, acc.at[dst_v], add=True)
            return carry

        lax.fori_loop(0, N_CHUNKS, chunk, 0)
        plsc.subcore_barrier()

        for k in range(ROWS_PER_TILE // CHUNK):
            pltpu.sync_copy(acc.at[pl.ds(r0 + k * CHUNK, CHUNK)], rows_v)
            pltpu.sync_copy(rows_v, out.at[c, pl.ds(r0 + k * CHUNK, CHUNK)])

    return pl.kernel(
        body,
        out_type=jax.ShapeDtypeStruct((NC, N_PAD, D), jnp.float32),
        mesh=_mesh,
        scratch_types=[
            pltpu.VMEM_SHARED((N_PAD, D), jnp.float32),
            pltpu.VMEM((CHUNK,), jnp.int32),
            pltpu.VMEM((CHUNK,), jnp.int32),
            pltpu.VMEM((CHUNK, D), jnp.float32),
            pltpu.SemaphoreType.DMA,
        ],
    )(table, src, dst, onesz)


def _deg_sum(dst, onesz):
    def body(dst, onesz, out, acc, dst_v, ones_v, rows_v):
        c = lax.axis_index("c")
        s = lax.axis_index("s")
        wid = s * NC + c
        r0 = s * ROWS_PER_TILE
        base0 = wid * PER_W

        for k in range(ROWS_PER_TILE // CHUNK):
            pltpu.sync_copy(onesz.at[pl.ds(CHUNK, CHUNK)], rows_v)
            pltpu.sync_copy(rows_v, acc.at[pl.ds(r0 + k * CHUNK, CHUNK)])
        pltpu.sync_copy(onesz.at[pl.ds(0, CHUNK)], ones_v)
        plsc.subcore_barrier()

        def chunk(i, carry):
            base = base0 + i * CHUNK
            pltpu.sync_copy(dst.at[pl.ds(base, CHUNK)], dst_v)
            pltpu.sync_copy(ones_v, acc.at[dst_v], add=True)
            return carry

        lax.fori_loop(0, N_CHUNKS, chunk, 0)
        plsc.subcore_barrier()

        for k in range(ROWS_PER_TILE // CHUNK):
            pltpu.sync_copy(acc.at[pl.ds(r0 + k * CHUNK, CHUNK)], rows_v)
            pltpu.sync_copy(rows_v, out.at[c, pl.ds(r0 + k * CHUNK, CHUNK)])

    return pl.kernel(
        body,
        out_type=jax.ShapeDtypeStruct((NC, N_PAD, D), jnp.float32),
        mesh=_mesh,
        scratch_types=[
            pltpu.VMEM_SHARED((N_PAD, D), jnp.float32),
            pltpu.VMEM((CHUNK,), jnp.int32),
            pltpu.VMEM((CHUNK, D), jnp.float32),
            pltpu.VMEM((CHUNK, D), jnp.float32),
        ],
    )(dst, onesz)


BLK = 512


def _tc_layer(parts, degparts, xin, WlT, bl, WrT):
    def body(p_ref, dg_ref, x_ref, wl_ref, bl_ref, wr_ref, o_ref):
        agg = p_ref[0] + p_ref[1]
        deg = dg_ref[0, :, 0:1] + dg_ref[1, :, 0:1]
        mean = agg / jnp.maximum(deg, 1.0)
        h = (jnp.dot(mean, wl_ref[...], preferred_element_type=jnp.float32)
             + bl_ref[...]
             + jnp.dot(x_ref[...], wr_ref[...], preferred_element_type=jnp.float32))
        o_ref[...] = jnp.maximum(h, 0.0)

    return pl.pallas_call(
        body,
        grid=(N_PAD // BLK,),
        in_specs=[
            pl.BlockSpec((2, BLK, D), lambda i: (0, i, 0)),
            pl.BlockSpec((2, BLK, D), lambda i: (0, i, 0)),
            pl.BlockSpec((BLK, D), lambda i: (i, 0)),
            pl.BlockSpec((D, D), lambda i: (0, 0)),
            pl.BlockSpec((1, D), lambda i: (0, 0)),
            pl.BlockSpec((D, D), lambda i: (0, 0)),
        ],
        out_specs=pl.BlockSpec((BLK, D), lambda i: (i, 0)),
        out_shape=jax.ShapeDtypeStruct((N_PAD, D), jnp.float32),
    )(parts, degparts, xin, WlT, bl, WrT)


def _tc_layer2_heads(parts, degparts, hin, WlT, bl, WrT, WpT, bp, WdT, bd):
    def body(p_ref, dg_ref, h_ref, wl_ref, bl_ref, wr_ref,
             wp_ref, bp_ref, wd_ref, bd_ref, lo_ref, up_ref):
        agg = p_ref[0] + p_ref[1]
        deg = dg_ref[0, :, 0:1] + dg_ref[1, :, 0:1]
        mean = agg / jnp.maximum(deg, 1.0)
        h2 = (jnp.dot(mean, wl_ref[...], preferred_element_type=jnp.float32)
              + bl_ref[...]
              + jnp.dot(h_ref[...], wr_ref[...], preferred_element_type=jnp.float32))
        h2 = jnp.maximum(h2, 0.0)
        p = jnp.dot(h2, wp_ref[...], preferred_element_type=jnp.float32) + bp_ref[...]
        dz = jnp.dot(h2, wd_ref[...], preferred_element_type=jnp.float32) + bd_ref[...]
        sg = 1.0 / (1.0 + jnp.exp(-dz))
        lo_ref[...] = p - sg
        up_ref[...] = p + sg

    return pl.pallas_call(
        body,
        grid=(N_PAD // BLK,),
        in_specs=[
            pl.BlockSpec((2, BLK, D), lambda i: (0, i, 0)),
            pl.BlockSpec((2, BLK, D), lambda i: (0, i, 0)),
            pl.BlockSpec((BLK, D), lambda i: (i, 0)),
            pl.BlockSpec((D, D), lambda i: (0, 0)),
            pl.BlockSpec((1, D), lambda i: (0, 0)),
            pl.BlockSpec((D, D), lambda i: (0, 0)),
            pl.BlockSpec((D, 1), lambda i: (0, 0)),
            pl.BlockSpec((1, 1), lambda i: (0, 0)),
            pl.BlockSpec((D, 1), lambda i: (0, 0)),
            pl.BlockSpec((1, 1), lambda i: (0, 0)),
        ],
        out_specs=[
            pl.BlockSpec((BLK, 1), lambda i: (i, 0)),
            pl.BlockSpec((BLK, 1), lambda i: (i, 0)),
        ],
        out_shape=[
            jax.ShapeDtypeStruct((N_PAD, 1), jnp.float32),
            jax.ShapeDtypeStruct((N_PAD, 1), jnp.float32),
        ],
    )(parts, degparts, hin, WlT, bl, WrT, WpT, bp, WdT, bd)


def kernel(x, edge_index, W1l, b1l, W1r, W2l, b2l, W2r, Wp, bp, Wd, bd):
    src = edge_index[0]
    dst = edge_index[1]
    pad_n = E_PAD - E
    pad_ids = jnp.arange(pad_n, dtype=jnp.int32)
    src_p = jnp.concatenate([src, pad_ids % N])
    dst_p = jnp.concatenate([dst, N + pad_ids % (N_PAD - N)])
    x_pad = jnp.zeros((N_PAD, D), jnp.float32).at[:N].set(x)
    onesz = jnp.concatenate([jnp.ones((CHUNK, D), jnp.float32),
                             jnp.zeros((CHUNK, D), jnp.float32)])

    degp = _deg_sum(dst_p, onesz)
    parts1 = _seg_sum(x_pad, src_p, dst_p, onesz)
    h1 = _tc_layer(parts1, degp, x_pad, W1l.T, b1l.reshape(1, D), W1r.T)
    parts2 = _seg_sum(h1, src_p, dst_p, onesz)
    lo, up = _tc_layer2_heads(parts2, degp, h1, W2l.T, b2l.reshape(1, D),
                              W2r.T, Wp.T, bp.reshape(1, 1),
                              Wd.T, bd.reshape(1, 1))
    return lo[:N], up[:N]


# speedup vs baseline: 10.1511x; 1.9236x over previous
import jax
import jax.numpy as jnp
from jax import lax
from jax.experimental import pallas as pl
from jax.experimental.pallas import tpu as pltpu
from jax.experimental.pallas import tpu_sc as plsc

N = 10000
E = 320000
D = 128

NC = 2
NS = 16
NW = NC * NS

N_PAD = 10240
ROWS_PER_TILE = N_PAD // NS
CHUNK = 128
N_CHUNKS = ((-(-E // (NW * CHUNK)) + 1) // 2) * 2
PER_W = N_CHUNKS * CHUNK
E_PAD = PER_W * NW

_mesh = plsc.VectorSubcoreMesh(core_axis_name="c", subcore_axis_name="s")


def _seg_sum(table, src, dst3, onesz):
    def body(table, src, dst3, onesz, out,
             acc, dst_v, rows0, rows1, s0, s1, g0, g1, l0, l1):
        c = lax.axis_index("c")
        s = lax.axis_index("s")
        wid = s * NC + c
        r0 = s * ROWS_PER_TILE
        base0 = wid * PER_W

        pltpu.sync_copy(dst3.at[wid], dst_v)

        pltpu.sync_copy(onesz.at[pl.ds(CHUNK, CHUNK)], rows0)
        for k in range(ROWS_PER_TILE // CHUNK):
            pltpu.sync_copy(rows0, acc.at[pl.ds(r0 + k * CHUNK, CHUNK)])
        plsc.subcore_barrier()

        def load_src(i, buf, sem):
            pltpu.async_copy(src.at[pl.ds(base0 + i * CHUNK, CHUNK)], buf, sem)

        def wait_src(buf, sem):
            pltpu.make_async_copy(src.at[pl.ds(base0, CHUNK)], buf, sem).wait()

        def gather(sbuf, rows, sem):
            pltpu.async_copy(table.at[sbuf], rows, sem)

        def scatter(i, rows, sem):
            pltpu.make_async_copy(table.at[s0], rows, sem).wait()
            pltpu.sync_copy(rows, acc.at[dst_v.at[i]], add=True)

        pltpu.sync_copy(src.at[pl.ds(base0, CHUNK)], s0)
        gather(s0, rows0, g0)
        load_src(1, s1, l1)

        @pl.loop(0, N_CHUNKS - 2, step=2)
        def pipe(i):
            wait_src(s1, l1)
            gather(s1, rows1, g1)
            load_src(i + 2, s0, l0)
            scatter(i, rows0, g0)
            wait_src(s0, l0)
            gather(s0, rows0, g0)
            load_src(i + 3, s1, l1)
            scatter(i + 1, rows1, g1)
        wait_src(s1, l1)
        gather(s1, rows1, g1)
        scatter(N_CHUNKS - 2, rows0, g0)
        scatter(N_CHUNKS - 1, rows1, g1)
        plsc.subcore_barrier()

        for k in range(ROWS_PER_TILE // CHUNK):
            pltpu.sync_copy(acc.at[pl.ds(r0 + k * CHUNK, CHUNK)], rows0)
            pltpu.sync_copy(rows0, out.at[c, pl.ds(r0 + k * CHUNK, CHUNK)])

    return pl.kernel(
        body,
        out_type=jax.ShapeDtypeStruct((NC, N_PAD, D), jnp.float32),
        mesh=_mesh,
        scratch_types=[
            pltpu.VMEM_SHARED((N_PAD, D), jnp.float32),
            pltpu.VMEM((N_CHUNKS, CHUNK), jnp.int32),
            pltpu.VMEM((CHUNK, D), jnp.float32),
            pltpu.VMEM((CHUNK, D), jnp.float32),
            pltpu.VMEM((CHUNK,), jnp.int32),
            pltpu.VMEM((CHUNK,), jnp.int32),
            pltpu.SemaphoreType.DMA,
            pltpu.SemaphoreType.DMA,
            pltpu.SemaphoreType.DMA,
            pltpu.SemaphoreType.DMA,
        ],
    )(table, src, dst3, onesz)


DW = 128


def _deg_sum(dst3, onesz32):
    def body(dst3, onesz32, out, acc, dst_v, ones_v, rows_v, sem0, sem1):
        c = lax.axis_index("c")
        s = lax.axis_index("s")
        wid = s * NC + c
        r0 = s * ROWS_PER_TILE

        pltpu.sync_copy(dst3.at[wid], dst_v)
        pltpu.sync_copy(onesz32.at[pl.ds(CHUNK, CHUNK)], rows_v)
        for k in range(ROWS_PER_TILE // CHUNK):
            pltpu.sync_copy(rows_v, acc.at[pl.ds(r0 + k * CHUNK, CHUNK)])
        pltpu.sync_copy(onesz32.at[pl.ds(0, CHUNK)], ones_v)
        plsc.subcore_barrier()

        def start(i, sem):
            pltpu.async_copy(ones_v, acc.at[dst_v.at[i]], sem, add=True)

        def drain(sem):
            pltpu.make_async_copy(ones_v, acc.at[dst_v.at[0]], sem).wait()

        start(0, sem0)

        @pl.loop(0, N_CHUNKS - 2, step=2)
        def pipe(i):
            start(i + 1, sem1)
            drain(sem0)
            start(i + 2, sem0)
            drain(sem1)
        start(N_CHUNKS - 1, sem1)
        drain(sem0)
        drain(sem1)
        plsc.subcore_barrier()

        for k in range(ROWS_PER_TILE // CHUNK):
            pltpu.sync_copy(acc.at[pl.ds(r0 + k * CHUNK, CHUNK)], rows_v)
            pltpu.sync_copy(rows_v, out.at[c, pl.ds(r0 + k * CHUNK, CHUNK)])

    return pl.kernel(
        body,
        out_type=jax.ShapeDtypeStruct((NC, N_PAD, DW), jnp.float32),
        mesh=_mesh,
        scratch_types=[
            pltpu.VMEM_SHARED((N_PAD, DW), jnp.float32),
            pltpu.VMEM((N_CHUNKS, CHUNK), jnp.int32),
            pltpu.VMEM((CHUNK, DW), jnp.float32),
            pltpu.VMEM((CHUNK, DW), jnp.float32),
            pltpu.SemaphoreType.DMA,
            pltpu.SemaphoreType.DMA,
        ],
    )(dst3, onesz32)


BLK = 400


def _tc_layer(parts, degparts, xin, WlT, bl, WrT):
    def body(p_ref, dg_ref, x_ref, wl_ref, bl_ref, wr_ref, o_ref):
        agg = p_ref[0] + p_ref[1]
        deg = dg_ref[0, :, 0:1] + dg_ref[1, :, 0:1]
        mean = agg / jnp.maximum(deg, 1.0)
        h = (jnp.dot(mean, wl_ref[...], preferred_element_type=jnp.float32)
             + bl_ref[...]
             + jnp.dot(x_ref[...], wr_ref[...], preferred_element_type=jnp.float32))
        o_ref[...] = jnp.maximum(h, 0.0)

    return pl.pallas_call(
        body,
        grid=(N // BLK,),
        in_specs=[
            pl.BlockSpec((2, BLK, D), lambda i: (0, i, 0)),
            pl.BlockSpec((2, BLK, DW), lambda i: (0, i, 0)),
            pl.BlockSpec((BLK, D), lambda i: (i, 0)),
            pl.BlockSpec((D, D), lambda i: (0, 0)),
            pl.BlockSpec((1, D), lambda i: (0, 0)),
            pl.BlockSpec((D, D), lambda i: (0, 0)),
        ],
        out_specs=pl.BlockSpec((BLK, D), lambda i: (i, 0)),
        out_shape=jax.ShapeDtypeStruct((N, D), jnp.float32),
    )(parts, degparts, xin, WlT, bl, WrT)


def _tc_layer2_heads(parts, degparts, hin, WlT, bl, WrT, WpT, bp, WdT, bd):
    def body(p_ref, dg_ref, h_ref, wl_ref, bl_ref, wr_ref,
             wp_ref, bp_ref, wd_ref, bd_ref, lo_ref, up_ref):
        agg = p_ref[0] + p_ref[1]
        deg = dg_ref[0, :, 0:1] + dg_ref[1, :, 0:1]
        mean = agg / jnp.maximum(deg, 1.0)
        h2 = (jnp.dot(mean, wl_ref[...], preferred_element_type=jnp.float32)
              + bl_ref[...]
              + jnp.dot(h_ref[...], wr_ref[...], preferred_element_type=jnp.float32))
        h2 = jnp.maximum(h2, 0.0)
        p = jnp.dot(h2, wp_ref[...], preferred_element_type=jnp.float32) + bp_ref[...]
        dz = jnp.dot(h2, wd_ref[...], preferred_element_type=jnp.float32) + bd_ref[...]
        sg = 1.0 / (1.0 + jnp.exp(-dz))
        lo_ref[...] = p - sg
        up_ref[...] = p + sg

    return pl.pallas_call(
        body,
        grid=(N // BLK,),
        in_specs=[
            pl.BlockSpec((2, BLK, D), lambda i: (0, i, 0)),
            pl.BlockSpec((2, BLK, DW), lambda i: (0, i, 0)),
            pl.BlockSpec((BLK, D), lambda i: (i, 0)),
            pl.BlockSpec((D, D), lambda i: (0, 0)),
            pl.BlockSpec((1, D), lambda i: (0, 0)),
            pl.BlockSpec((D, D), lambda i: (0, 0)),
            pl.BlockSpec((D, 1), lambda i: (0, 0)),
            pl.BlockSpec((1, 1), lambda i: (0, 0)),
            pl.BlockSpec((D, 1), lambda i: (0, 0)),
            pl.BlockSpec((1, 1), lambda i: (0, 0)),
        ],
        out_specs=[
            pl.BlockSpec((BLK, 1), lambda i: (i, 0)),
            pl.BlockSpec((BLK, 1), lambda i: (i, 0)),
        ],
        out_shape=[
            jax.ShapeDtypeStruct((N, 1), jnp.float32),
            jax.ShapeDtypeStruct((N, 1), jnp.float32),
        ],
    )(parts, degparts, hin, WlT, bl, WrT, WpT, bp, WdT, bd)


def kernel(x, edge_index, W1l, b1l, W1r, W2l, b2l, W2r, Wp, bp, Wd, bd):
    src = edge_index[0]
    dst = edge_index[1]
    pad_n = E_PAD - E
    pad_ids = jnp.arange(pad_n, dtype=jnp.int32)
    src_p = jnp.concatenate([src, pad_ids % N])
    dst_p = jnp.concatenate([dst, N + pad_ids % (N_PAD - N)])
    dst3 = dst_p.reshape(NW, N_CHUNKS, CHUNK)
    onesz = jnp.concatenate([jnp.ones((CHUNK, D), jnp.float32),
                             jnp.zeros((CHUNK, D), jnp.float32)])
    onesz32 = jnp.concatenate([jnp.ones((CHUNK, DW), jnp.float32),
                               jnp.zeros((CHUNK, DW), jnp.float32)])

    degp = _deg_sum(dst3, onesz32)
    parts1 = _seg_sum(x, src_p, dst3, onesz)
    h1 = _tc_layer(parts1, degp, x, W1l.T, b1l.reshape(1, D), W1r.T)
    parts2 = _seg_sum(h1, src_p, dst3, onesz)
    lo, up = _tc_layer2_heads(parts2, degp, h1, W2l.T, b2l.reshape(1, D),
                              W2r.T, Wp.T, bp.reshape(1, 1),
                              Wd.T, bd.reshape(1, 1))
    return lo, up


# speedup vs baseline: 10.1592x; 1.0008x over previous
import jax
import jax.numpy as jnp
from jax import lax
from jax.experimental import pallas as pl
from jax.experimental.pallas import tpu as pltpu
from jax.experimental.pallas import tpu_sc as plsc

N = 10000
E = 320000
D = 128

NC = 2
NS = 16
NW = NC * NS

N_PAD = 10240
ROWS_PER_TILE = N_PAD // NS
CHUNK = 128
N_CHUNKS = ((-(-E // (NW * CHUNK)) + 1) // 2) * 2
PER_W = N_CHUNKS * CHUNK
E_PAD = PER_W * NW

_mesh = plsc.VectorSubcoreMesh(core_axis_name="c", subcore_axis_name="s")


def _seg_sum(table, src, dst3, onesz):
    def body(table, src, dst3, onesz, out,
             acc, dst_v, rows0, rows1, s0, s1, g0, g1, l0, l1):
        c = lax.axis_index("c")
        s = lax.axis_index("s")
        wid = s * NC + c
        r0 = s * ROWS_PER_TILE
        base0 = wid * PER_W

        pltpu.sync_copy(dst3.at[wid], dst_v)

        pltpu.sync_copy(onesz.at[pl.ds(CHUNK, CHUNK)], rows0)
        for k in range(ROWS_PER_TILE // CHUNK):
            pltpu.sync_copy(rows0, acc.at[pl.ds(r0 + k * CHUNK, CHUNK)])
        plsc.subcore_barrier()

        def load_src(i, buf, sem):
            pltpu.async_copy(src.at[pl.ds(base0 + i * CHUNK, CHUNK)], buf, sem)

        def wait_src(buf, sem):
            pltpu.make_async_copy(src.at[pl.ds(base0, CHUNK)], buf, sem).wait()

        def gather(sbuf, rows, sem):
            pltpu.async_copy(table.at[sbuf], rows, sem)

        def scatter(i, rows, sem):
            pltpu.make_async_copy(table.at[s0], rows, sem).wait()
            pltpu.sync_copy(rows, acc.at[dst_v.at[i]], add=True)

        pltpu.sync_copy(src.at[pl.ds(base0, CHUNK)], s0)
        gather(s0, rows0, g0)
        load_src(1, s1, l1)

        @pl.loop(0, N_CHUNKS - 2, step=2)
        def pipe(i):
            wait_src(s1, l1)
            gather(s1, rows1, g1)
            load_src(i + 2, s0, l0)
            scatter(i, rows0, g0)
            wait_src(s0, l0)
            gather(s0, rows0, g0)
            load_src(i + 3, s1, l1)
            scatter(i + 1, rows1, g1)
        wait_src(s1, l1)
        gather(s1, rows1, g1)
        scatter(N_CHUNKS - 2, rows0, g0)
        scatter(N_CHUNKS - 1, rows1, g1)
        plsc.subcore_barrier()

        for k in range(ROWS_PER_TILE // CHUNK):
            pltpu.sync_copy(acc.at[pl.ds(r0 + k * CHUNK, CHUNK)], rows0)
            pltpu.sync_copy(rows0, out.at[c, pl.ds(r0 + k * CHUNK, CHUNK)])

    return pl.kernel(
        body,
        out_type=jax.ShapeDtypeStruct((NC, N_PAD, D), jnp.float32),
        mesh=_mesh,
        scratch_types=[
            pltpu.VMEM_SHARED((N_PAD, D), jnp.float32),
            pltpu.VMEM((N_CHUNKS, CHUNK), jnp.int32),
            pltpu.VMEM((CHUNK, D), jnp.float32),
            pltpu.VMEM((CHUNK, D), jnp.float32),
            pltpu.VMEM((CHUNK,), jnp.int32),
            pltpu.VMEM((CHUNK,), jnp.int32),
            pltpu.SemaphoreType.DMA,
            pltpu.SemaphoreType.DMA,
            pltpu.SemaphoreType.DMA,
            pltpu.SemaphoreType.DMA,
        ],
    )(table, src, dst3, onesz)


DW = 128


def _deg_sum(dst3, onesz32):
    def body(dst3, onesz32, out, acc, dst_v, ones_v, rows_v, sem0, sem1):
        c = lax.axis_index("c")
        s = lax.axis_index("s")
        wid = s * NC + c
        r0 = s * ROWS_PER_TILE

        pltpu.sync_copy(dst3.at[wid], dst_v)
        pltpu.sync_copy(onesz32.at[pl.ds(CHUNK, CHUNK)], rows_v)
        for k in range(ROWS_PER_TILE // CHUNK):
            pltpu.sync_copy(rows_v, acc.at[pl.ds(r0 + k * CHUNK, CHUNK)])
        pltpu.sync_copy(onesz32.at[pl.ds(0, CHUNK)], ones_v)
        plsc.subcore_barrier()

        def start(i, sem):
            pltpu.async_copy(ones_v, acc.at[dst_v.at[i]], sem, add=True)

        def drain(sem):
            pltpu.make_async_copy(ones_v, acc.at[dst_v.at[0]], sem).wait()

        start(0, sem0)
        start(1, sem1)
        start(2, sem0)
        start(3, sem1)

        @pl.loop(0, N_CHUNKS - 4, step=2)
        def pipe(i):
            drain(sem0)
            start(i + 4, sem0)
            drain(sem1)
            start(i + 5, sem1)
        drain(sem0)
        drain(sem1)
        drain(sem0)
        drain(sem1)
        plsc.subcore_barrier()

        for k in range(ROWS_PER_TILE // CHUNK):
            pltpu.sync_copy(acc.at[pl.ds(r0 + k * CHUNK, CHUNK)], rows_v)
            pltpu.sync_copy(rows_v, out.at[c, pl.ds(r0 + k * CHUNK, CHUNK)])

    return pl.kernel(
        body,
        out_type=jax.ShapeDtypeStruct((NC, N_PAD, DW), jnp.float32),
        mesh=_mesh,
        scratch_types=[
            pltpu.VMEM_SHARED((N_PAD, DW), jnp.float32),
            pltpu.VMEM((N_CHUNKS, CHUNK), jnp.int32),
            pltpu.VMEM((CHUNK, DW), jnp.float32),
            pltpu.VMEM((CHUNK, DW), jnp.float32),
            pltpu.SemaphoreType.DMA,
            pltpu.SemaphoreType.DMA,
        ],
    )(dst3, onesz32)


BLK = 400


def _tc_layer(parts, degparts, xin, WlT, bl, WrT):
    def body(p_ref, dg_ref, x_ref, wl_ref, bl_ref, wr_ref, o_ref):
        agg = p_ref[0] + p_ref[1]
        deg = dg_ref[0, :, 0:1] + dg_ref[1, :, 0:1]
        mean = agg / jnp.maximum(deg, 1.0)
        h = (jnp.dot(mean, wl_ref[...], preferred_element_type=jnp.float32)
             + bl_ref[...]
             + jnp.dot(x_ref[...], wr_ref[...], preferred_element_type=jnp.float32))
        o_ref[...] = jnp.maximum(h, 0.0)

    return pl.pallas_call(
        body,
        grid=(N // BLK,),
        in_specs=[
            pl.BlockSpec((2, BLK, D), lambda i: (0, i, 0)),
            pl.BlockSpec((2, BLK, DW), lambda i: (0, i, 0)),
            pl.BlockSpec((BLK, D), lambda i: (i, 0)),
            pl.BlockSpec((D, D), lambda i: (0, 0)),
            pl.BlockSpec((1, D), lambda i: (0, 0)),
            pl.BlockSpec((D, D), lambda i: (0, 0)),
        ],
        out_specs=pl.BlockSpec((BLK, D), lambda i: (i, 0)),
        out_shape=jax.ShapeDtypeStruct((N, D), jnp.float32),
    )(parts, degparts, xin, WlT, bl, WrT)


def _tc_layer2_heads(parts, degparts, hin, WlT, bl, WrT, WpT, bp, WdT, bd):
    def body(p_ref, dg_ref, h_ref, wl_ref, bl_ref, wr_ref,
             wp_ref, bp_ref, wd_ref, bd_ref, lo_ref, up_ref):
        agg = p_ref[0] + p_ref[1]
        deg = dg_ref[0, :, 0:1] + dg_ref[1, :, 0:1]
        mean = agg / jnp.maximum(deg, 1.0)
        h2 = (jnp.dot(mean, wl_ref[...], preferred_element_type=jnp.float32)
              + bl_ref[...]
              + jnp.dot(h_ref[...], wr_ref[...], preferred_element_type=jnp.float32))
        h2 = jnp.maximum(h2, 0.0)
        p = jnp.dot(h2, wp_ref[...], preferred_element_type=jnp.float32) + bp_ref[...]
        dz = jnp.dot(h2, wd_ref[...], preferred_element_type=jnp.float32) + bd_ref[...]
        sg = 1.0 / (1.0 + jnp.exp(-dz))
        lo_ref[...] = p - sg
        up_ref[...] = p + sg

    return pl.pallas_call(
        body,
        grid=(N // BLK,),
        in_specs=[
            pl.BlockSpec((2, BLK, D), lambda i: (0, i, 0)),
            pl.BlockSpec((2, BLK, DW), lambda i: (0, i, 0)),
            pl.BlockSpec((BLK, D), lambda i: (i, 0)),
            pl.BlockSpec((D, D), lambda i: (0, 0)),
            pl.BlockSpec((1, D), lambda i: (0, 0)),
            pl.BlockSpec((D, D), lambda i: (0, 0)),
            pl.BlockSpec((D, 1), lambda i: (0, 0)),
            pl.BlockSpec((1, 1), lambda i: (0, 0)),
            pl.BlockSpec((D, 1), lambda i: (0, 0)),
            pl.BlockSpec((1, 1), lambda i: (0, 0)),
        ],
        out_specs=[
            pl.BlockSpec((BLK, 1), lambda i: (i, 0)),
            pl.BlockSpec((BLK, 1), lambda i: (i, 0)),
        ],
        out_shape=[
            jax.ShapeDtypeStruct((N, 1), jnp.float32),
            jax.ShapeDtypeStruct((N, 1), jnp.float32),
        ],
    )(parts, degparts, hin, WlT, bl, WrT, WpT, bp, WdT, bd)


def kernel(x, edge_index, W1l, b1l, W1r, W2l, b2l, W2r, Wp, bp, Wd, bd):
    src = edge_index[0]
    dst = edge_index[1]
    pad_n = E_PAD - E
    pad_ids = jnp.arange(pad_n, dtype=jnp.int32)
    src_p = jnp.concatenate([src, pad_ids % N])
    dst_p = jnp.concatenate([dst, N + pad_ids % (N_PAD - N)])
    dst3 = dst_p.reshape(NW, N_CHUNKS, CHUNK)
    onesz = jnp.concatenate([jnp.ones((CHUNK, D), jnp.float32),
                             jnp.zeros((CHUNK, D), jnp.float32)])
    onesz32 = jnp.concatenate([jnp.ones((CHUNK, DW), jnp.float32),
                               jnp.zeros((CHUNK, DW), jnp.float32)])

    degp = _deg_sum(dst3, onesz32)
    parts1 = _seg_sum(x, src_p, dst3, onesz)
    h1 = _tc_layer(parts1, degp, x, W1l.T, b1l.reshape(1, D), W1r.T)
    parts2 = _seg_sum(h1, src_p, dst3, onesz)
    lo, up = _tc_layer2_heads(parts2, degp, h1, W2l.T, b2l.reshape(1, D),
                              W2r.T, Wp.T, bp.reshape(1, 1),
                              Wd.T, bd.reshape(1, 1))
    return lo, up
